# baseline (device time: 192858 ns/iter reference)
import jax
import jax.numpy as jnp
from jax import lax
from jax.experimental import pallas as pl
from jax.experimental.pallas import tpu as pltpu

N_DEV = 16
S = 8


def kernel(x, w_mat):
    m, k_sh = x.shape
    _, n = w_mat.shape
    chunk = m // N_DEV
    half = n // 2
    sub = half // S

    def body(x_ref, w_ref, out_ref, comm_ref,
             rs_send, rs_recv, ag_send, ag_recv):
        p = lax.axis_index("i")
        left = (p - 1) % N_DEV
        right = (p + 1) % N_DEV

        barrier_sem = pltpu.get_barrier_semaphore()
        for nbr in (left, right):
            pl.semaphore_signal(barrier_sem, inc=1, device_id=(nbr,),
                                device_id_type=pl.DeviceIdType.MESH)

        def gemm_chunk(c):
            rows = pl.ds(c * chunk, chunk)
            out_ref[rows, :] = jnp.dot(x_ref[rows, :], w_ref[:, :],
                                       preferred_element_type=jnp.float32)

        gemm_chunk(p)
        pl.semaphore_wait(barrier_sem, 2)

        def rs_copy(h, d, s_i):
            c_send = (p - h) % N_DEV if d == 0 else (p + h) % N_DEV
            tgt = right if d == 0 else left
            co = d * half + s_i * sub
            return pltpu.make_async_remote_copy(
                src_ref=out_ref.at[pl.ds(c_send * chunk, chunk),
                                   pl.ds(co, sub)],
                dst_ref=comm_ref.at[h, d, s_i],
                send_sem=rs_send.at[h, d, s_i],
                recv_sem=rs_recv.at[h, d, s_i],
                device_id=(tgt,),
                device_id_type=pl.DeviceIdType.MESH,
            )

        def ag_copy(t, d, s_i):
            c_send = (p + 1 - t) % N_DEV if d == 0 else (p - 1 + t) % N_DEV
            tgt = right if d == 0 else left
            ref = out_ref.at[pl.ds(c_send * chunk, chunk),
                             pl.ds(d * half + s_i * sub, sub)]
            return pltpu.make_async_remote_copy(
                src_ref=ref,
                dst_ref=ref,
                send_sem=ag_send.at[t, d, s_i],
                recv_sem=ag_recv.at[t, d, s_i],
                device_id=(tgt,),
                device_id_type=pl.DeviceIdType.MESH,
            )

        for d in range(2):
            for s_i in range(S):
                rs_copy(0, d, s_i).start()
        for j in range(1, N_DEV // 2 + 1):
            gemm_chunk((p - j) % N_DEV)
            if j < N_DEV // 2:
                gemm_chunk((p + j) % N_DEV)
        c_gelu = 0.7978845608028654
        for h in range(N_DEV - 1):
            for s_i in range(S):
                for d in range(2):
                    c_recv = (p - h - 1) % N_DEV if d == 0 else (p + h + 1) % N_DEV
                    rows = pl.ds(c_recv * chunk, chunk)
                    r = rs_copy(h, d, s_i)
                    r.wait_recv()
                    cols = pl.ds(d * half + s_i * sub, sub)
                    acc = out_ref[rows, cols] + comm_ref[h, d, s_i]
                    if h + 1 < N_DEV - 1:
                        out_ref[rows, cols] = acc
                        rs_copy(h + 1, d, s_i).start()
                    else:
                        out_ref[rows, cols] = 0.5 * acc * (
                            1.0 + jnp.tanh(c_gelu * (acc + 0.044715 * acc * acc * acc)))
                        ag_copy(0, d, s_i).start()
                    r.wait_send()

        for t in range(N_DEV - 1):
            for s_i in range(S):
                for d in range(2):
                    r = ag_copy(t, d, s_i)
                    r.wait_recv()
                    if t + 1 < N_DEV - 1:
                        ag_copy(t + 1, d, s_i).start()
                    r.wait_send()

    return pl.pallas_call(
        body,
        out_shape=jax.ShapeDtypeStruct((m, n), jnp.float32),
        in_specs=[
            pl.BlockSpec(memory_space=pltpu.VMEM),
            pl.BlockSpec(memory_space=pltpu.VMEM),
        ],
        out_specs=pl.BlockSpec(memory_space=pltpu.VMEM),
        scratch_shapes=[
            pltpu.VMEM((N_DEV - 1, 2, S, chunk, sub), jnp.float32),
            pltpu.SemaphoreType.DMA((N_DEV - 1, 2, S)),
            pltpu.SemaphoreType.DMA((N_DEV - 1, 2, S)),
            pltpu.SemaphoreType.DMA((N_DEV - 1, 2, S)),
            pltpu.SemaphoreType.DMA((N_DEV - 1, 2, S)),
        ],
        compiler_params=pltpu.CompilerParams(collective_id=0),
    )(x, w_mat)


# device time: 188589 ns/iter; 1.0226x vs baseline; 1.0226x over previous
import jax
import jax.numpy as jnp
from jax import lax
from jax.experimental import pallas as pl
from jax.experimental.pallas import tpu as pltpu

N_DEV = 16
S = 4


def kernel(x, w_mat):
    m, k_sh = x.shape
    _, n = w_mat.shape
    chunk = m // N_DEV
    half = n // 2
    sub = half // S

    def body(x_ref, w_ref, out_ref, comm_ref,
             rs_send, rs_recv, ag_send, ag_recv):
        p = lax.axis_index("i")
        left = (p - 1) % N_DEV
        right = (p + 1) % N_DEV

        barrier_sem = pltpu.get_barrier_semaphore()
        for nbr in (left, right):
            pl.semaphore_signal(barrier_sem, inc=1, device_id=(nbr,),
                                device_id_type=pl.DeviceIdType.MESH)

        def gemm_chunk(c):
            rows = pl.ds(c * chunk, chunk)
            out_ref[rows, :] = jnp.dot(x_ref[rows, :], w_ref[:, :],
                                       preferred_element_type=jnp.float32)

        gemm_chunk(p)
        pl.semaphore_wait(barrier_sem, 2)

        def rs_copy(h, d, s_i):
            c_send = (p - h) % N_DEV if d == 0 else (p + h) % N_DEV
            tgt = right if d == 0 else left
            co = d * half + s_i * sub
            return pltpu.make_async_remote_copy(
                src_ref=out_ref.at[pl.ds(c_send * chunk, chunk),
                                   pl.ds(co, sub)],
                dst_ref=comm_ref.at[h, d, s_i],
                send_sem=rs_send.at[h, d, s_i],
                recv_sem=rs_recv.at[h, d, s_i],
                device_id=(tgt,),
                device_id_type=pl.DeviceIdType.MESH,
            )

        def ag_copy(t, d, s_i):
            c_send = (p + 1 - t) % N_DEV if d == 0 else (p - 1 + t) % N_DEV
            tgt = right if d == 0 else left
            ref = out_ref.at[pl.ds(c_send * chunk, chunk),
                             pl.ds(d * half + s_i * sub, sub)]
            return pltpu.make_async_remote_copy(
                src_ref=ref,
                dst_ref=ref,
                send_sem=ag_send.at[t, d, s_i],
                recv_sem=ag_recv.at[t, d, s_i],
                device_id=(tgt,),
                device_id_type=pl.DeviceIdType.MESH,
            )

        for d in range(2):
            for s_i in range(S):
                rs_copy(0, d, s_i).start()
        for j in range(1, N_DEV // 2 + 1):
            gemm_chunk((p - j) % N_DEV)
            if j < N_DEV // 2:
                gemm_chunk((p + j) % N_DEV)
        c_gelu = 0.7978845608028654
        for h in range(N_DEV - 1):
            for s_i in range(S):
                for d in range(2):
                    c_recv = (p - h - 1) % N_DEV if d == 0 else (p + h + 1) % N_DEV
                    rows = pl.ds(c_recv * chunk, chunk)
                    r = rs_copy(h, d, s_i)
                    r.wait_recv()
                    cols = pl.ds(d * half + s_i * sub, sub)
                    acc = out_ref[rows, cols] + comm_ref[h, d, s_i]
                    if h + 1 < N_DEV - 1:
                        out_ref[rows, cols] = acc
                        rs_copy(h + 1, d, s_i).start()
                    else:
                        out_ref[rows, cols] = 0.5 * acc * (
                            1.0 + jnp.tanh(c_gelu * (acc + 0.044715 * acc * acc * acc)))
                        ag_copy(0, d, s_i).start()
                    r.wait_send()

        for t in range(N_DEV - 1):
            for s_i in range(S):
                for d in range(2):
                    r = ag_copy(t, d, s_i)
                    r.wait_recv()
                    if t + 1 < N_DEV - 1:
                        ag_copy(t + 1, d, s_i).start()
                    r.wait_send()

    return pl.pallas_call(
        body,
        out_shape=jax.ShapeDtypeStruct((m, n), jnp.float32),
        in_specs=[
            pl.BlockSpec(memory_space=pltpu.VMEM),
            pl.BlockSpec(memory_space=pltpu.VMEM),
        ],
        out_specs=pl.BlockSpec(memory_space=pltpu.VMEM),
        scratch_shapes=[
            pltpu.VMEM((N_DEV - 1, 2, S, chunk, sub), jnp.float32),
            pltpu.SemaphoreType.DMA((N_DEV - 1, 2, S)),
            pltpu.SemaphoreType.DMA((N_DEV - 1, 2, S)),
            pltpu.SemaphoreType.DMA((N_DEV - 1, 2, S)),
            pltpu.SemaphoreType.DMA((N_DEV - 1, 2, S)),
        ],
        compiler_params=pltpu.CompilerParams(collective_id=0),
    )(x, w_mat)
